# baseline (speedup 1.0000x reference)
import functools

import jax
import jax.numpy as jnp
from jax import lax
from jax.experimental import pallas as pl
from jax.experimental.pallas import tpu as pltpu
from jax.experimental.pallas import tpu_sc as plsc

L = 16
NC, NS = 2, 16
NW = NC * NS
BATCH = 16384
FIELDS = 26
BPW = BATCH // NW
KPW = BPW * FIELDS
IDX_MINOR = 128
ROWS = KPW // IDX_MINOR


def _sc_call(xt, table, bias, *, interpret=False):
    mesh = plsc.VectorSubcoreMesh(
        core_axis_name="c", subcore_axis_name="s", num_cores=NC, num_subcores=NS
    )

    @functools.partial(
        pl.kernel,
        out_type=jax.ShapeDtypeStruct((BATCH,), jnp.float32),
        mesh=mesh,
        scratch_types=[
            pltpu.VMEM((KPW,), jnp.int32),
            pltpu.VMEM((KPW,), jnp.float32),
            pltpu.VMEM((BPW,), jnp.float32),
            pltpu.VMEM((L,), jnp.float32),
            pltpu.SemaphoreType.DMA,
        ],
        compiler_params=pltpu.CompilerParams(needs_layout_passes=False),
        interpret=interpret,
    )
    def k(x_hbm, table_hbm, bias_hbm, out_hbm, idx_v, val_v, out_v, bias_v, sem):
        wid = lax.axis_index("s") * NC + lax.axis_index("c")
        pltpu.sync_copy(x_hbm.at[wid], idx_v)
        pltpu.sync_copy(bias_hbm, bias_v)
        pltpu.async_copy(table_hbm.at[idx_v], val_v, sem).wait()
        bvec = bias_v[...]
        lanes = lax.iota(jnp.int32, L)
        for c in range(BPW // L):
            acc = bvec
            p = (lanes + c * L) * FIELDS
            for _ in range(FIELDS):
                acc = acc + plsc.load_gather(val_v, [p])
                p = p + 1
            out_v[pl.ds(c * L, L)] = acc
        pltpu.sync_copy(out_v, out_hbm.at[pl.ds(wid * BPW, BPW)])

    return k(xt, table, bias)


def kernel(x, fc_weight, bias):
    xt = x.reshape(NW, KPW)
    table = fc_weight.reshape(-1)
    bias16 = jnp.broadcast_to(bias.reshape(()), (L,))
    out = _sc_call(xt, table, bias16)
    return out.reshape(BATCH, 1)


# speedup vs baseline: 1.1119x; 1.1119x over previous
import functools

import jax
import jax.numpy as jnp
from jax import lax
from jax.experimental import pallas as pl
from jax.experimental.pallas import tpu as pltpu
from jax.experimental.pallas import tpu_sc as plsc

L = 16
NC, NS = 2, 16
NW = NC * NS
BATCH = 16384
FIELDS = 26
BPW = BATCH // NW
KPW = BPW * FIELDS
IDX_MINOR = 128
ROWS = KPW // IDX_MINOR


def _sc_call(xt, table, bias, *, interpret=False):
    mesh = plsc.VectorSubcoreMesh(
        core_axis_name="c", subcore_axis_name="s", num_cores=NC, num_subcores=NS
    )

    @functools.partial(
        pl.kernel,
        out_type=jax.ShapeDtypeStruct((BATCH,), jnp.float32),
        mesh=mesh,
        scratch_types=[
            pltpu.VMEM((KPW,), jnp.int32),
            pltpu.VMEM((KPW,), jnp.float32),
            pltpu.VMEM((BPW,), jnp.float32),
            pltpu.VMEM((L,), jnp.float32),
            pltpu.SemaphoreType.DMA,
        ],
        compiler_params=pltpu.CompilerParams(disable_bounds_checks=True),
        interpret=interpret,
    )
    def k(x_hbm, table_hbm, bias_hbm, out_hbm, idx_v, val_v, out_v, bias_v, sem):
        wid = lax.axis_index("s") * NC + lax.axis_index("c")
        pltpu.sync_copy(x_hbm.at[wid], idx_v)
        pltpu.sync_copy(bias_hbm, bias_v)
        pltpu.async_copy(table_hbm.at[idx_v], val_v, sem).wait()
        bvec = bias_v[...]
        for c in range(BPW // L):
            acc = bvec
            for f in range(FIELDS):
                acc = acc + val_v[pl.ds(f * BPW + c * L, L)]
            out_v[pl.ds(c * L, L)] = acc
        pltpu.sync_copy(out_v, out_hbm.at[pl.ds(wid * BPW, BPW)])

    return k(xt, table, bias)


def kernel(x, fc_weight, bias):
    xt = (
        x.reshape(NW, BPW, FIELDS)
        .transpose(0, 2, 1)
        .reshape(NW, KPW)
    )
    table = fc_weight.reshape(-1)
    bias16 = jnp.broadcast_to(bias.reshape(()), (L,))
    out = _sc_call(xt, table, bias16)
    return out.reshape(BATCH, 1)


# speedup vs baseline: 2.7124x; 2.4393x over previous
import functools

import jax
import jax.numpy as jnp
from jax import lax
from jax.experimental import pallas as pl
from jax.experimental.pallas import tpu as pltpu
from jax.experimental.pallas import tpu_sc as plsc

L = 16
NC, NS = 2, 16
NW = NC * NS
BATCH = 16384
FIELDS = 26
VOCAB = 1000000
BPW = BATCH // NW
KPW = BPW * FIELDS
STAGERS = 8
CHUNK = 124928
TAIL_OFF = STAGERS * CHUNK
TAIL = VOCAB - TAIL_OFF


def _sc_call(xt, table, bias16, *, interpret=False):
    mesh = plsc.VectorSubcoreMesh(
        core_axis_name="c", subcore_axis_name="s", num_cores=NC, num_subcores=NS
    )

    @functools.partial(
        pl.kernel,
        out_type=jax.ShapeDtypeStruct((BATCH,), jnp.float32),
        mesh=mesh,
        scratch_types=[
            pltpu.VMEM_SHARED((VOCAB,), jnp.float32),
            pltpu.VMEM((KPW,), jnp.int32),
            pltpu.VMEM((KPW,), jnp.float32),
            pltpu.VMEM((BPW,), jnp.float32),
            pltpu.VMEM((L,), jnp.float32),
            pltpu.SemaphoreType.DMA,
        ],
        compiler_params=pltpu.CompilerParams(disable_bounds_checks=True),
        interpret=interpret,
    )
    def k(x_hbm, table_hbm, bias_hbm, out_hbm,
          table_sh, idx_v, val_v, out_v, bias_v, sem):
        cid = lax.axis_index("c")
        sid = lax.axis_index("s")
        wid = sid * NC + cid
        pltpu.sync_copy(x_hbm.at[wid], idx_v)
        pltpu.sync_copy(bias_hbm, bias_v)

        @pl.when(sid < STAGERS)
        def _stage():
            pltpu.sync_copy(
                table_hbm.at[0, pl.ds(sid * CHUNK, CHUNK)],
                table_sh.at[pl.ds(sid * CHUNK, CHUNK)],
            )

        @pl.when(sid == STAGERS)
        def _stage_tail():
            pltpu.sync_copy(
                table_hbm.at[0, pl.ds(TAIL_OFF, TAIL)], val_v.at[pl.ds(0, TAIL)]
            )
            pltpu.sync_copy(
                val_v.at[pl.ds(0, TAIL)], table_sh.at[pl.ds(TAIL_OFF, TAIL)]
            )

        plsc.subcore_barrier()
        pltpu.async_copy(table_sh.at[idx_v], val_v, sem).wait()
        bvec = bias_v[...]
        for c in range(BPW // L):
            acc = bvec
            for f in range(FIELDS):
                acc = acc + val_v[pl.ds(f * BPW + c * L, L)]
            out_v[pl.ds(c * L, L)] = acc
        pltpu.sync_copy(out_v, out_hbm.at[pl.ds(wid * BPW, BPW)])

    return k(xt, table, bias16)


def kernel(x, fc_weight, bias):
    xt = (
        x.reshape(NW, BPW, FIELDS)
        .transpose(0, 2, 1)
        .reshape(NW, KPW)
    )
    bias16 = jnp.broadcast_to(bias.reshape(()), (L,))
    table = fc_weight.T
    out = _sc_call(xt, table, bias16)
    return out.reshape(BATCH, 1)


# speedup vs baseline: 2.8839x; 1.0632x over previous
import functools

import jax
import jax.numpy as jnp
from jax import lax
from jax.experimental import pallas as pl
from jax.experimental.pallas import tpu as pltpu
from jax.experimental.pallas import tpu_sc as plsc

L = 16
NC, NS = 2, 16
NW = NC * NS
BATCH = 16384
FIELDS = 26
VOCAB = 1000000
BPW = BATCH // NW
KPW = BPW * FIELDS
F_HALF = FIELDS // 2
KPH = BPW * F_HALF
CHUNK = 62464
TAIL_OFF = NS * CHUNK
TAIL = VOCAB - TAIL_OFF


def _sc_call(xt, table, bias16, *, interpret=False):
    mesh = plsc.VectorSubcoreMesh(
        core_axis_name="c", subcore_axis_name="s", num_cores=NC, num_subcores=NS
    )

    @functools.partial(
        pl.kernel,
        out_type=jax.ShapeDtypeStruct((BATCH,), jnp.float32),
        mesh=mesh,
        scratch_types=[
            pltpu.VMEM_SHARED((VOCAB,), jnp.float32),
            pltpu.VMEM((KPW,), jnp.int32),
            pltpu.VMEM((KPW,), jnp.float32),
            pltpu.VMEM((BPW,), jnp.float32),
            pltpu.VMEM((L,), jnp.float32),
            pltpu.SemaphoreType.DMA,
            pltpu.SemaphoreType.DMA,
        ],
        compiler_params=pltpu.CompilerParams(disable_bounds_checks=True),
        interpret=interpret,
    )
    def k(x_hbm, table_hbm, bias_hbm, out_hbm,
          table_sh, idx_v, val_v, out_v, bias_v, sem_s, sem_g):
        cid = lax.axis_index("c")
        sid = lax.axis_index("s")
        wid = sid * NC + cid
        stage = pltpu.async_copy(
            table_hbm.at[0, pl.ds(sid * CHUNK, CHUNK)],
            table_sh.at[pl.ds(sid * CHUNK, CHUNK)],
            sem_s,
        )
        pltpu.sync_copy(x_hbm.at[wid], idx_v)
        pltpu.sync_copy(bias_hbm, bias_v)

        @pl.when(sid == 0)
        def _stage_tail():
            pltpu.sync_copy(
                table_hbm.at[0, pl.ds(TAIL_OFF, TAIL)], val_v.at[pl.ds(0, TAIL)]
            )
            pltpu.sync_copy(
                val_v.at[pl.ds(0, TAIL)], table_sh.at[pl.ds(TAIL_OFF, TAIL)]
            )

        stage.wait()
        plsc.subcore_barrier()
        g0 = pltpu.async_copy(
            table_sh.at[idx_v.at[pl.ds(0, KPH)]], val_v.at[pl.ds(0, KPH)], sem_g
        )
        g1 = pltpu.async_copy(
            table_sh.at[idx_v.at[pl.ds(KPH, KPH)]],
            val_v.at[pl.ds(KPH, KPH)],
            sem_g,
        )
        bvec = bias_v[...]
        g0.wait()
        for c in range(BPW // L):
            acc = bvec
            for f in range(F_HALF):
                acc = acc + val_v[pl.ds(f * BPW + c * L, L)]
            out_v[pl.ds(c * L, L)] = acc
        g1.wait()
        for c in range(BPW // L):
            acc = out_v[pl.ds(c * L, L)]
            for f in range(F_HALF, FIELDS):
                acc = acc + val_v[pl.ds(f * BPW + c * L, L)]
            out_v[pl.ds(c * L, L)] = acc
        pltpu.sync_copy(out_v, out_hbm.at[pl.ds(wid * BPW, BPW)])

    return k(xt, table, bias16)


def kernel(x, fc_weight, bias):
    xt = (
        x.reshape(NW, BPW, FIELDS)
        .transpose(0, 2, 1)
        .reshape(NW, KPW)
    )
    bias16 = jnp.broadcast_to(bias.reshape(()), (L,))
    table = fc_weight.T
    out = _sc_call(xt, table, bias16)
    return out.reshape(BATCH, 1)


# speedup vs baseline: 2.9530x; 1.0239x over previous
import functools

import jax
import jax.numpy as jnp
from jax import lax
from jax.experimental import pallas as pl
from jax.experimental.pallas import tpu as pltpu
from jax.experimental.pallas import tpu_sc as plsc

L = 16
NC, NS = 2, 16
NW = NC * NS
BATCH = 16384
FIELDS = 26
VOCAB = 1000000
BPW = BATCH // NW
KPW = BPW * FIELDS
F_HALF = FIELDS // 2
KPH = BPW * F_HALF
CHUNK = 62464
TAIL_OFF = NS * CHUNK
TAIL = VOCAB - TAIL_OFF


def _sc_call(xt, table, bias16, *, interpret=False):
    mesh = plsc.VectorSubcoreMesh(
        core_axis_name="c", subcore_axis_name="s", num_cores=NC, num_subcores=NS
    )

    @functools.partial(
        pl.kernel,
        out_type=jax.ShapeDtypeStruct((BATCH,), jnp.float32),
        mesh=mesh,
        scratch_types=[
            pltpu.VMEM_SHARED((VOCAB,), jnp.float32),
            pltpu.VMEM((KPW,), jnp.int32),
            pltpu.VMEM((KPW,), jnp.float32),
            pltpu.VMEM((BPW,), jnp.float32),
            pltpu.VMEM((L,), jnp.float32),
            pltpu.SemaphoreType.DMA,
            pltpu.SemaphoreType.DMA,
            pltpu.SemaphoreType.DMA,
        ],
        compiler_params=pltpu.CompilerParams(disable_bounds_checks=True),
        interpret=interpret,
    )
    def k(x_hbm, table_hbm, bias_hbm, out_hbm,
          table_sh, idx_v, val_v, out_v, bias_v, sem_s, sem_g, sem_i):
        cid = lax.axis_index("c")
        sid = lax.axis_index("s")
        wid = sid * NC + cid
        stage = pltpu.async_copy(
            table_hbm.at[0, pl.ds(sid * CHUNK, CHUNK)],
            table_sh.at[pl.ds(sid * CHUNK, CHUNK)],
            sem_s,
        )
        col = pl.multiple_of(wid * BPW, 128)
        for f in range(FIELDS):
            pltpu.async_copy(
                x_hbm.at[f, pl.ds(col, BPW)],
                idx_v.at[pl.ds(f * BPW, BPW)],
                sem_i,
            )
        pltpu.sync_copy(bias_hbm, bias_v)

        @pl.when(sid == 0)
        def _stage_tail():
            pltpu.sync_copy(
                table_hbm.at[0, pl.ds(TAIL_OFF, TAIL)], val_v.at[pl.ds(0, TAIL)]
            )
            pltpu.sync_copy(
                val_v.at[pl.ds(0, TAIL)], table_sh.at[pl.ds(TAIL_OFF, TAIL)]
            )

        pltpu.make_async_copy(x_hbm.at[0, pl.ds(0, KPW)], idx_v, sem_i).wait()
        stage.wait()
        plsc.subcore_barrier()
        g0 = pltpu.async_copy(
            table_sh.at[idx_v.at[pl.ds(0, KPH)]], val_v.at[pl.ds(0, KPH)], sem_g
        )
        g1 = pltpu.async_copy(
            table_sh.at[idx_v.at[pl.ds(KPH, KPH)]],
            val_v.at[pl.ds(KPH, KPH)],
            sem_g,
        )
        bvec = bias_v[...]
        g0.wait()
        for c in range(BPW // L):
            acc = bvec
            for f in range(F_HALF):
                acc = acc + val_v[pl.ds(f * BPW + c * L, L)]
            out_v[pl.ds(c * L, L)] = acc
        g1.wait()
        for c in range(BPW // L):
            acc = out_v[pl.ds(c * L, L)]
            for f in range(F_HALF, FIELDS):
                acc = acc + val_v[pl.ds(f * BPW + c * L, L)]
            out_v[pl.ds(c * L, L)] = acc
        pltpu.sync_copy(out_v, out_hbm.at[pl.ds(wid * BPW, BPW)])

    return k(xt, table, bias16)


def kernel(x, fc_weight, bias):
    xt = x.T
    bias16 = jnp.broadcast_to(bias.reshape(()), (L,))
    table = fc_weight.T
    out = _sc_call(xt, table, bias16)
    return out.reshape(BATCH, 1)


# speedup vs baseline: 2.9806x; 1.0094x over previous
import functools

import jax
import jax.numpy as jnp
from jax import lax
from jax.experimental import pallas as pl
from jax.experimental.pallas import tpu as pltpu
from jax.experimental.pallas import tpu_sc as plsc

L = 16
NC, NS = 2, 16
NW = NC * NS
BATCH = 16384
FIELDS = 26
VOCAB = 1000000
BPW = BATCH // NW
KPW = BPW * FIELDS
QUARTERS = ((0, 7), (7, 13), (13, 20), (20, 26))
CHUNK = 62464
TAIL_OFF = NS * CHUNK
TAIL = VOCAB - TAIL_OFF


def _sc_call(xt, table, bias16, *, interpret=False):
    mesh = plsc.VectorSubcoreMesh(
        core_axis_name="c", subcore_axis_name="s", num_cores=NC, num_subcores=NS
    )

    @functools.partial(
        pl.kernel,
        out_type=jax.ShapeDtypeStruct((BATCH,), jnp.float32),
        mesh=mesh,
        scratch_types=[
            pltpu.VMEM_SHARED((VOCAB,), jnp.float32),
            pltpu.VMEM((KPW,), jnp.int32),
            pltpu.VMEM((KPW,), jnp.float32),
            pltpu.VMEM((BPW,), jnp.float32),
            pltpu.VMEM((L,), jnp.float32),
            pltpu.SemaphoreType.DMA,
            pltpu.SemaphoreType.DMA,
            pltpu.SemaphoreType.DMA,
        ],
        compiler_params=pltpu.CompilerParams(disable_bounds_checks=True),
        interpret=interpret,
    )
    def k(x_hbm, table_hbm, bias_hbm, out_hbm,
          table_sh, idx_v, val_v, out_v, bias_v, sem_s, sem_g, sem_i):
        cid = lax.axis_index("c")
        sid = lax.axis_index("s")
        wid = sid * NC + cid
        stage = pltpu.async_copy(
            table_hbm.at[0, pl.ds(sid * CHUNK, CHUNK)],
            table_sh.at[pl.ds(sid * CHUNK, CHUNK)],
            sem_s,
        )
        col = pl.multiple_of(wid * BPW, 128)
        for f in range(FIELDS):
            pltpu.async_copy(
                x_hbm.at[f, pl.ds(col, BPW)],
                idx_v.at[pl.ds(f * BPW, BPW)],
                sem_i,
            )
        pltpu.sync_copy(bias_hbm, bias_v)

        @pl.when(sid == 0)
        def _stage_tail():
            pltpu.sync_copy(
                table_hbm.at[0, pl.ds(TAIL_OFF, TAIL)], val_v.at[pl.ds(0, TAIL)]
            )
            pltpu.sync_copy(
                val_v.at[pl.ds(0, TAIL)], table_sh.at[pl.ds(TAIL_OFF, TAIL)]
            )

        pltpu.make_async_copy(x_hbm.at[0, pl.ds(0, KPW)], idx_v, sem_i).wait()
        stage.wait()
        plsc.subcore_barrier()
        gathers = []
        for f_lo, f_hi in QUARTERS:
            o, n = f_lo * BPW, (f_hi - f_lo) * BPW
            gathers.append(
                pltpu.async_copy(
                    table_sh.at[idx_v.at[pl.ds(o, n)]],
                    val_v.at[pl.ds(o, n)],
                    sem_g,
                )
            )
        bvec = bias_v[...]
        for q, (f_lo, f_hi) in enumerate(QUARTERS):
            gathers[q].wait()
            for c in range(BPW // L):
                acc = bvec if q == 0 else out_v[pl.ds(c * L, L)]
                for f in range(f_lo, f_hi):
                    acc = acc + val_v[pl.ds(f * BPW + c * L, L)]
                out_v[pl.ds(c * L, L)] = acc
        pltpu.sync_copy(out_v, out_hbm.at[pl.ds(wid * BPW, BPW)])

    return k(xt, table, bias16)


def kernel(x, fc_weight, bias):
    xt = x.T
    bias16 = jnp.broadcast_to(bias.reshape(()), (L,))
    table = fc_weight.T
    out = _sc_call(xt, table, bias16)
    return out.reshape(BATCH, 1)


# speedup vs baseline: 3.2073x; 1.0760x over previous
import functools

import jax
import jax.numpy as jnp
from jax import lax
from jax.experimental import pallas as pl
from jax.experimental.pallas import tpu as pltpu
from jax.experimental.pallas import tpu_sc as plsc

L = 16
NC, NS = 2, 16
NW = NC * NS
BATCH = 16384
FIELDS = 26
VOCAB = 1000000
BPW = BATCH // NW
KPW = BPW * FIELDS
QUARTERS = ((0, 7), (7, 13), (13, 20), (20, 26))
CHUNK = 62464
TAIL_OFF = NS * CHUNK
TAIL = VOCAB - TAIL_OFF


def _sc_call(xt, table, bias, *, interpret=False):
    mesh = plsc.VectorSubcoreMesh(
        core_axis_name="c", subcore_axis_name="s", num_cores=NC, num_subcores=NS
    )

    @functools.partial(
        pl.kernel,
        out_type=jax.ShapeDtypeStruct((BATCH,), jnp.float32),
        mesh=mesh,
        scratch_types=[
            pltpu.VMEM_SHARED((VOCAB,), jnp.float32),
            pltpu.VMEM((KPW,), jnp.int32),
            pltpu.VMEM((KPW,), jnp.float32),
            pltpu.VMEM((BPW,), jnp.float32),
            pltpu.VMEM((L,), jnp.float32),
            pltpu.SemaphoreType.DMA,
            pltpu.SemaphoreType.DMA,
            pltpu.SemaphoreType.DMA,
        ],
        compiler_params=pltpu.CompilerParams(
            disable_bounds_checks=True, needs_layout_passes=False
        ),
        interpret=interpret,
    )
    def k(x_hbm, table_hbm, bias_hbm, out_hbm,
          table_sh, idx_v, val_v, out_v, bias_v, sem_s, sem_g, sem_i):
        cid = lax.axis_index("c")
        sid = lax.axis_index("s")
        wid = sid * NC + cid
        stage = pltpu.async_copy(
            table_hbm.at[0, pl.ds(sid * CHUNK, CHUNK)],
            table_sh.at[pl.ds(sid * CHUNK, CHUNK)],
            sem_s,
        )
        col = pl.multiple_of(wid * BPW, 128)
        for f in range(FIELDS):
            pltpu.async_copy(
                x_hbm.at[f, pl.ds(col, BPW)],
                idx_v.at[pl.ds(f * BPW, BPW)],
                sem_i,
            )
        bias_v[...] = jnp.zeros((L,), jnp.float32)
        pltpu.sync_copy(bias_hbm, bias_v.at[pl.ds(0, 1)])

        @pl.when(sid == 0)
        def _stage_tail():
            pltpu.sync_copy(
                table_hbm.at[0, pl.ds(TAIL_OFF, TAIL)], val_v.at[pl.ds(0, TAIL)]
            )
            pltpu.sync_copy(
                val_v.at[pl.ds(0, TAIL)], table_sh.at[pl.ds(TAIL_OFF, TAIL)]
            )

        pltpu.make_async_copy(x_hbm.at[0, pl.ds(0, KPW)], idx_v, sem_i).wait()
        stage.wait()
        plsc.subcore_barrier()
        gathers = []
        for f_lo, f_hi in QUARTERS:
            o, n = f_lo * BPW, (f_hi - f_lo) * BPW
            gathers.append(
                pltpu.async_copy(
                    table_sh.at[idx_v.at[pl.ds(o, n)]],
                    val_v.at[pl.ds(o, n)],
                    sem_g,
                )
            )
        bvec = jnp.full((L,), jnp.sum(bias_v[...]), dtype=jnp.float32)
        for q, (f_lo, f_hi) in enumerate(QUARTERS):
            gathers[q].wait()

            def _chunk(c, _, q=q, f_lo=f_lo, f_hi=f_hi):
                o = c * L
                acc = bvec if q == 0 else out_v[pl.ds(o, L)]
                for f in range(f_lo, f_hi):
                    acc = acc + val_v[pl.ds(f * BPW + o, L)]
                out_v[pl.ds(o, L)] = acc
                return _

            lax.fori_loop(0, BPW // L, _chunk, None)
        pltpu.sync_copy(out_v, out_hbm.at[pl.ds(wid * BPW, BPW)])

    return k(xt, table, bias)


def kernel(x, fc_weight, bias):
    xt = x.T
    table = fc_weight.T
    out = _sc_call(xt, table, bias)
    return out.reshape(BATCH, 1)


# speedup vs baseline: 3.2159x; 1.0027x over previous
import functools

import jax
import jax.numpy as jnp
from jax import lax
from jax.experimental import pallas as pl
from jax.experimental.pallas import tpu as pltpu
from jax.experimental.pallas import tpu_sc as plsc

L = 16
NC, NS = 2, 16
NW = NC * NS
BATCH = 16384
FIELDS = 26
VOCAB = 1000000
BPW = BATCH // NW
KPW = BPW * FIELDS
QUARTERS = ((0, 7), (7, 13), (13, 20), (20, 26))
CHUNK = 62464
TAIL_OFF = NS * CHUNK
TAIL = VOCAB - TAIL_OFF


def _sc_call(xt, table, bias, *, interpret=False):
    mesh = plsc.VectorSubcoreMesh(
        core_axis_name="c", subcore_axis_name="s", num_cores=NC, num_subcores=NS
    )

    @functools.partial(
        pl.kernel,
        out_type=jax.ShapeDtypeStruct((BATCH,), jnp.float32),
        mesh=mesh,
        scratch_types=[
            pltpu.VMEM_SHARED((VOCAB,), jnp.float32),
            pltpu.VMEM((KPW,), jnp.int32),
            pltpu.VMEM((KPW,), jnp.float32),
            pltpu.VMEM((BPW,), jnp.float32),
            pltpu.VMEM((L,), jnp.float32),
            pltpu.SemaphoreType.DMA,
            pltpu.SemaphoreType.DMA,
            pltpu.SemaphoreType.DMA,
        ],
        compiler_params=pltpu.CompilerParams(
            disable_bounds_checks=True, needs_layout_passes=False
        ),
        interpret=interpret,
    )
    def k(x_hbm, table_hbm, bias_hbm, out_hbm,
          table_sh, idx_v, val_v, out_v, bias_v, sem_s, sem_g, sem_i):
        cid = lax.axis_index("c")
        sid = lax.axis_index("s")
        wid = sid * NC + cid
        stage = pltpu.async_copy(
            table_hbm.at[0, pl.ds(sid * CHUNK, CHUNK)],
            table_sh.at[pl.ds(sid * CHUNK, CHUNK)],
            sem_s,
        )
        col = pl.multiple_of(wid * BPW, 128)

        def _idx_dma(f, _):
            pltpu.async_copy(
                x_hbm.at[f, pl.ds(col, BPW)],
                idx_v.at[pl.ds(pl.multiple_of(f * BPW, 128), BPW)],
                sem_i,
            )
            return _

        lax.fori_loop(0, FIELDS, _idx_dma, None)
        bias_v[...] = jnp.zeros((L,), jnp.float32)
        pltpu.sync_copy(bias_hbm, bias_v.at[pl.ds(0, 1)])

        @pl.when(sid == 0)
        def _stage_tail():
            pltpu.sync_copy(
                table_hbm.at[0, pl.ds(TAIL_OFF, TAIL)], val_v.at[pl.ds(0, TAIL)]
            )
            pltpu.sync_copy(
                val_v.at[pl.ds(0, TAIL)], table_sh.at[pl.ds(TAIL_OFF, TAIL)]
            )

        pltpu.make_async_copy(x_hbm.at[0, pl.ds(0, KPW)], idx_v, sem_i).wait()
        stage.wait()
        plsc.subcore_barrier()
        gathers = []
        for f_lo, f_hi in QUARTERS:
            o, n = f_lo * BPW, (f_hi - f_lo) * BPW
            gathers.append(
                pltpu.async_copy(
                    table_sh.at[idx_v.at[pl.ds(o, n)]],
                    val_v.at[pl.ds(o, n)],
                    sem_g,
                )
            )
        bvec = jnp.full((L,), jnp.sum(bias_v[...]), dtype=jnp.float32)
        for q, (f_lo, f_hi) in enumerate(QUARTERS):
            gathers[q].wait()

            def _chunk(c, _, q=q, f_lo=f_lo, f_hi=f_hi):
                o = c * L
                acc = bvec if q == 0 else out_v[pl.ds(o, L)]
                for f in range(f_lo, f_hi):
                    acc = acc + val_v[pl.ds(f * BPW + o, L)]
                out_v[pl.ds(o, L)] = acc
                return _

            lax.fori_loop(0, BPW // L, _chunk, None)
        pltpu.sync_copy(out_v, out_hbm.at[pl.ds(wid * BPW, BPW)])

    return k(xt, table, bias)


def kernel(x, fc_weight, bias):
    xt = x.T
    table = fc_weight.T
    out = _sc_call(xt, table, bias)
    return out.reshape(BATCH, 1)


# speedup vs baseline: 3.2239x; 1.0025x over previous
import functools

import jax
import jax.numpy as jnp
from jax import lax
from jax.experimental import pallas as pl
from jax.experimental.pallas import tpu as pltpu
from jax.experimental.pallas import tpu_sc as plsc

L = 16
NC, NS = 2, 16
NW = NC * NS
BATCH = 16384
FIELDS = 26
VOCAB = 1000000
BPW = BATCH // NW
KPW = BPW * FIELDS
QUARTERS = ((0, 7), (7, 13), (13, 20), (20, 26))
CHUNK = 62464
TAIL_OFF = NS * CHUNK
TAIL = VOCAB - TAIL_OFF


def _sc_call(xt, table, bias):
    mesh = plsc.VectorSubcoreMesh(
        core_axis_name="c", subcore_axis_name="s", num_cores=NC, num_subcores=NS
    )

    @functools.partial(
        pl.kernel,
        out_type=jax.ShapeDtypeStruct((BATCH,), jnp.float32),
        mesh=mesh,
        scratch_types=[
            pltpu.VMEM_SHARED((VOCAB,), jnp.float32),
            pltpu.VMEM((KPW,), jnp.int32),
            pltpu.VMEM((KPW,), jnp.float32),
            pltpu.VMEM((BPW,), jnp.float32),
            pltpu.VMEM((L,), jnp.float32),
            pltpu.SemaphoreType.DMA,
            pltpu.SemaphoreType.DMA,
            pltpu.SemaphoreType.DMA,
        ],
        compiler_params=pltpu.CompilerParams(
            disable_bounds_checks=True, needs_layout_passes=False
        ),
    )
    def k(x_hbm, table_hbm, bias_hbm, out_hbm,
          table_sh, idx_v, val_v, out_v, bias_v, sem_s, sem_g, sem_i):
        cid = lax.axis_index("c")
        sid = lax.axis_index("s")
        wid = sid * NC + cid
        stage = pltpu.async_copy(
            table_hbm.at[0, pl.ds(sid * CHUNK, CHUNK)],
            table_sh.at[pl.ds(sid * CHUNK, CHUNK)],
            sem_s,
        )
        col = pl.multiple_of(wid * BPW, 128)

        def _idx_dma(f, _):
            pltpu.async_copy(
                x_hbm.at[f, pl.ds(col, BPW)],
                idx_v.at[pl.ds(pl.multiple_of(f * BPW, 128), BPW)],
                sem_i,
            )
            return _

        lax.fori_loop(0, FIELDS, _idx_dma, None)
        bias_v[...] = jnp.zeros((L,), jnp.float32)
        pltpu.sync_copy(bias_hbm, bias_v.at[pl.ds(0, 1)])

        @pl.when(sid == 0)
        def _stage_tail():
            pltpu.sync_copy(
                table_hbm.at[0, pl.ds(TAIL_OFF, TAIL)], val_v.at[pl.ds(0, TAIL)]
            )
            pltpu.sync_copy(
                val_v.at[pl.ds(0, TAIL)], table_sh.at[pl.ds(TAIL_OFF, TAIL)]
            )

        pltpu.make_async_copy(x_hbm.at[0, pl.ds(0, KPW)], idx_v, sem_i).wait()
        stage.wait()
        plsc.subcore_barrier()
        gathers = []
        for f_lo, f_hi in QUARTERS:
            o, n = f_lo * BPW, (f_hi - f_lo) * BPW
            gathers.append(
                pltpu.async_copy(
                    table_sh.at[idx_v.at[pl.ds(o, n)]],
                    val_v.at[pl.ds(o, n)],
                    sem_g,
                )
            )
        bvec = jnp.full((L,), jnp.sum(bias_v[...]), dtype=jnp.float32)
        for q, (f_lo, f_hi) in enumerate(QUARTERS):
            gathers[q].wait()

            def _chunk(c, _, q=q, f_lo=f_lo, f_hi=f_hi):
                o = c * L
                acc = bvec if q == 0 else out_v[pl.ds(o, L)]
                for f in range(f_lo, f_hi):
                    acc = acc + val_v[pl.ds(f * BPW + o, L)]
                out_v[pl.ds(o, L)] = acc
                return _

            lax.fori_loop(0, BPW // L, _chunk, None)
        pltpu.sync_copy(out_v, out_hbm.at[pl.ds(wid * BPW, BPW)])

    return k(xt, table, bias)


def kernel(x, fc_weight, bias):
    xt = x.T
    table = fc_weight.T
    out = _sc_call(xt, table, bias)
    return out.reshape(BATCH, 1)
